# Initial kernel scaffold; baseline (speedup 1.0000x reference)
import jax
import jax.numpy as jnp
from jax.experimental import pallas as pl


def kernel(x, judge_ids, W1_w, W1_b, W2_w, W2_b, W1a_w, W1a_b, W2a_w, W2a_b, V_w, V_b, Va_w, Va_b):
    raise NotImplementedError("write your pallas kernel here")


# speedup vs baseline: 4.6357x; 4.6357x over previous
import jax
import jax.numpy as jnp
from jax.experimental import pallas as pl


def _fused_body(x_ref, jid_ref, w1t_ref, b1_ref, w2t_ref, b2_ref,
                w1at_ref, b1a_ref, w2at_ref, b2a_ref,
                vwt_ref, vb_ref, vawt_ref, vab_ref, out_ref):
    x = x_ref[...]
    jid = jid_ref[...]
    B, D = x.shape
    w1t = w1t_ref[...]
    w2t = w2t_ref[...]
    vwt = vwt_ref[...]
    b1 = b1_ref[...]
    b2 = b2_ref[...]
    vb = vb_ref[...]
    J = w1at_ref.shape[0]
    H1 = w1t.shape[1]
    H2 = w2t.shape[1]
    QC = vwt.shape[1]
    f32 = jnp.float32

    def body(j, logits_acc):
        m = (jid == j).astype(f32)
        w1 = w1at_ref[j]
        bb1 = b1a_ref[j]
        w2 = w2at_ref[j]
        bb2 = b2a_ref[j]
        vw = vawt_ref[j]
        vvb = vab_ref[j]
        w1c = w1t + w1
        w2c = w2t + w2
        vwc = vwt + vw
        xm = x * m
        z1 = jnp.maximum(
            jnp.dot(xm, w1c[:D], preferred_element_type=f32)
            + m * (w1c[D:D + 1] + b1 + bb1), 0.0)
        z2 = jnp.maximum(
            jnp.dot(z1, w2c[:H1], preferred_element_type=f32)
            + m * (w2c[H1:H1 + 1] + b2 + bb2), 0.0)
        lg = (jnp.dot(z2, vwc[:H2], preferred_element_type=f32)
              + m * (vwc[H2:H2 + 1] + vb + vvb))
        return logits_acc + lg

    logits = jax.lax.fori_loop(0, J, body, jnp.zeros((B, QC), f32))
    mx = jnp.max(logits, axis=1, keepdims=True)
    e = jnp.exp(logits - mx)
    C = 5
    gi = jax.lax.broadcasted_iota(jnp.int32, (QC, QC), 0) // C
    gj = jax.lax.broadcasted_iota(jnp.int32, (QC, QC), 1) // C
    grp = (gi == gj).astype(f32)
    denom = jnp.dot(e, grp, preferred_element_type=f32)
    out_ref[...] = e / denom


def kernel(x, judge_ids, W1_w, W1_b, W2_w, W2_b, W1a_w, W1a_b, W2a_w, W2a_b,
           V_w, V_b, Va_w, Va_b):
    B, D = x.shape
    J, H1, _ = W1a_w.shape
    H2 = W2a_w.shape[1]
    Q, C, _ = V_w.shape
    QC = Q * C
    jid = judge_ids.astype(jnp.int32).reshape(B, 1)
    w1t = W1_w.T
    w2t = W2_w.T
    vwt = V_w.reshape(QC, H2 + 1).T
    w1at = W1a_w.transpose(0, 2, 1)
    w2at = W2a_w.transpose(0, 2, 1)
    vawt = Va_w.reshape(J, QC, H2 + 1).transpose(0, 2, 1)
    b1 = W1_b.reshape(1, H1)
    b2 = W2_b.reshape(1, H2)
    vb = V_b.reshape(1, QC)
    b1a = W1a_b.reshape(J, 1, H1)
    b2a = W2a_b.reshape(J, 1, H2)
    vab = Va_b.reshape(J, 1, QC)

    out = pl.pallas_call(
        _fused_body,
        out_shape=jax.ShapeDtypeStruct((B, QC), jnp.float32),
    )(x, jid, w1t, b1, w2t, b2, w1at, b1a, w2at, b2a, vwt, vb, vawt, vab)
    return out.reshape(B, Q, C).transpose(1, 0, 2)


# speedup vs baseline: 6.6452x; 1.4335x over previous
import functools

import jax
import jax.numpy as jnp
from jax import lax
from jax.experimental import pallas as pl
from jax.experimental.pallas import tpu as pltpu
from jax.experimental.pallas import tpu_sc as plsc

R = 64
G = 8
OUTW = 128
_SCB = 1


def _dispatch_body(B, J, SB, NW, NC, jid_hbm, x_hbm,
                   xs_hbm, pos_hbm, end_hbm,
                   jid_v, hist_v, rank_v, base_v, end_v,
                   pos_v, rows_v, sem):
    wid = lax.axis_index("s") * NC + lax.axis_index("c")
    i32 = jnp.int32

    pltpu.sync_copy(jid_hbm, jid_v)

    for k in range(J // 16):
        hist_v[pl.ds(k * 16, 16)] = jnp.zeros((16,), i32)

    def rank_step(k, c):
        sl = pl.ds(k * 16, 16)
        jv = jid_v[sl]
        before = plsc.load_gather(hist_v, [jv])
        cnt, last = plsc.scan_count(jv)
        rank_v[sl] = before + cnt - _SCB
        plsc.store_scatter(hist_v, [jv], before + cnt + (1 - _SCB), mask=last)
        return c

    lax.fori_loop(0, B // 16, rank_step, jnp.zeros((), i32))

    carry = jnp.zeros((), i32)
    for k in range(J // 16):
        sl = pl.ds(k * 16, 16)
        padded = ((hist_v[sl] + (R - 1)) // R) * R
        incl = plsc.cumsum(padded)
        excl = incl - padded + carry
        base_v[sl] = excl
        end_v[sl] = excl + padded
        carry = carry + jnp.sum(padded)

    @pl.when(wid == 0)
    def _():
        pltpu.sync_copy(end_v, end_hbm)

    for k in range(SB // 16):
        sl = pl.ds(k * 16, 16)
        g = wid * SB + k * 16
        jv = jid_v[pl.ds(g, 16)]
        pos_v[sl] = plsc.load_gather(base_v, [jv]) + rank_v[pl.ds(g, 16)]
    pltpu.sync_copy(pos_v, pos_hbm.at[pl.ds(wid * SB, SB)])

    pltpu.sync_copy(x_hbm.at[pl.ds(wid * SB, SB)], rows_v)
    pltpu.async_copy(rows_v, xs_hbm.at[pos_v], sem).wait()


def _unsort_body(SB, NC, osort_hbm, pos_hbm, out_hbm, pos_v, rows_v, sem):
    wid = lax.axis_index("s") * NC + lax.axis_index("c")
    base = wid * SB
    pltpu.sync_copy(pos_hbm.at[pl.ds(base, SB)], pos_v)
    pltpu.async_copy(osort_hbm.at[pos_v], rows_v, sem).wait()
    pltpu.sync_copy(rows_v, out_hbm.at[pl.ds(base, SB)])


def _segment_body(D, H1, H2, QC, x_ref, end_ref, w1t_ref, b1c_ref, w2t_ref,
                  b2c_ref, vwt_ref, vbc_ref, out_ref):
    f32 = jnp.float32
    gg = pl.program_id(0)
    ends = end_ref[...]
    gi = lax.broadcasted_iota(jnp.int32, (QC, QC), 0) // 5
    gj = lax.broadcasted_iota(jnp.int32, (QC, QC), 1) // 5
    grp = (gi == gj).astype(f32)
    for sb in range(G):
        s = (gg * G + sb) * R
        bj = jnp.minimum(jnp.sum((ends <= s).astype(jnp.int32)),
                         end_ref.shape[1] - 1)
        w1 = w1t_ref[bj]
        bb1 = b1c_ref[bj]
        w2 = w2t_ref[bj]
        bb2 = b2c_ref[bj]
        vw = vwt_ref[bj]
        vvb = vbc_ref[bj]
        x = x_ref[sb * R:(sb + 1) * R]
        z1 = jnp.maximum(jnp.dot(x, w1[:D], preferred_element_type=f32)
                         + (w1[D:D + 1] + bb1), 0.0)
        z2 = jnp.maximum(jnp.dot(z1, w2[:H1], preferred_element_type=f32)
                         + (w2[H1:H1 + 1] + bb2), 0.0)
        lg = (jnp.dot(z2, vw[:H2], preferred_element_type=f32)
              + (vw[H2:H2 + 1] + vvb))
        mx = jnp.max(lg, axis=1, keepdims=True)
        e = jnp.exp(lg - mx)
        denom = jnp.dot(e, grp, preferred_element_type=f32)
        sm = e / denom
        out_ref[sb * R:(sb + 1) * R] = jnp.concatenate(
            [sm, jnp.zeros((sm.shape[0], OUTW - QC), f32)], axis=1)


def kernel(x, judge_ids, W1_w, W1_b, W2_w, W2_b, W1a_w, W1a_b, W2a_w, W2a_b,
           V_w, V_b, Va_w, Va_b):
    B, D = x.shape
    J, H1, _ = W1a_w.shape
    H2 = W2a_w.shape[1]
    Q, C, _ = V_w.shape
    QC = Q * C
    Bpad = 2 * B
    NB = Bpad // R

    info = plsc.get_sparse_core_info()
    NC, NS = info.num_cores, info.num_subcores
    NW = NC * NS
    SB = B // NW
    i32 = jnp.int32
    f32 = jnp.float32

    jid = judge_ids.astype(i32)
    w1t = W1_w.T[None] + W1a_w.transpose(0, 2, 1)
    w2t = W2_w.T[None] + W2a_w.transpose(0, 2, 1)
    vwt = (V_w.reshape(QC, H2 + 1).T[None]
           + Va_w.reshape(J, QC, H2 + 1).transpose(0, 2, 1))
    b1c = (W1_b[None] + W1a_b).reshape(J, 1, H1)
    b2c = (W2_b[None] + W2a_b).reshape(J, 1, H2)
    vbc = (V_b.reshape(1, QC) + Va_b.reshape(J, QC)).reshape(J, 1, QC)

    mesh = plsc.VectorSubcoreMesh(core_axis_name="c", subcore_axis_name="s")

    dispatch = pl.kernel(
        functools.partial(_dispatch_body, B, J, SB, NW, NC),
        out_type=(jax.ShapeDtypeStruct((Bpad, D), f32),
                  jax.ShapeDtypeStruct((B,), i32),
                  jax.ShapeDtypeStruct((J,), i32)),
        mesh=mesh,
        compiler_params=pltpu.CompilerParams(needs_layout_passes=False),
        scratch_types=[
            pltpu.VMEM((B,), i32),
            pltpu.VMEM((J,), i32),
            pltpu.VMEM((B,), i32),
            pltpu.VMEM((J,), i32),
            pltpu.VMEM((J,), i32),
            pltpu.VMEM((SB,), i32),
            pltpu.VMEM((SB, D), f32),
            pltpu.SemaphoreType.DMA,
        ],
    )
    x_sorted, pos, end = dispatch(jid, x)

    out_sorted = pl.pallas_call(
        functools.partial(_segment_body, D, H1, H2, QC),
        grid=(NB // G,),
        in_specs=[
            pl.BlockSpec((G * R, D), lambda g: (g, 0)),
            pl.BlockSpec((1, J), lambda g: (0, 0)),
            pl.BlockSpec((J, D + 1, H1), lambda g: (0, 0, 0)),
            pl.BlockSpec((J, 1, H1), lambda g: (0, 0, 0)),
            pl.BlockSpec((J, H1 + 1, H2), lambda g: (0, 0, 0)),
            pl.BlockSpec((J, 1, H2), lambda g: (0, 0, 0)),
            pl.BlockSpec((J, H2 + 1, QC), lambda g: (0, 0, 0)),
            pl.BlockSpec((J, 1, QC), lambda g: (0, 0, 0)),
        ],
        out_specs=pl.BlockSpec((G * R, OUTW), lambda g: (g, 0)),
        out_shape=jax.ShapeDtypeStruct((Bpad, OUTW), f32),
    )(x_sorted, end.reshape(1, J), w1t, b1c, w2t, b2c, vwt, vbc)

    unsort = pl.kernel(
        functools.partial(_unsort_body, SB, NC),
        out_type=jax.ShapeDtypeStruct((B, OUTW), f32),
        mesh=mesh,
        compiler_params=pltpu.CompilerParams(needs_layout_passes=False),
        scratch_types=[
            pltpu.VMEM((SB,), i32),
            pltpu.VMEM((SB, OUTW), f32),
            pltpu.SemaphoreType.DMA,
        ],
    )
    out_full = unsort(out_sorted, pos)
    return out_full[:, :QC].reshape(B, Q, C).transpose(1, 0, 2)


# speedup vs baseline: 10.3157x; 1.5524x over previous
import functools

import jax
import jax.numpy as jnp
from jax import lax
from jax.experimental import pallas as pl
from jax.experimental.pallas import tpu as pltpu
from jax.experimental.pallas import tpu_sc as plsc

R = 64
G = 8
OUTW = 128
_SCB = 1

_DN_T = (((1,), (1,)), ((), ()))


def _dispatch_body(B, J, SB, NW, NC, jid_hbm, x_hbm,
                   xs_hbm, pos_hbm, end_hbm,
                   jid_v, hist_v, rank_v, base_v, end_v,
                   pos_v, rows_v, sem):
    wid = lax.axis_index("s") * NC + lax.axis_index("c")
    i32 = jnp.int32

    pltpu.sync_copy(jid_hbm, jid_v)

    for k in range(J // 16):
        hist_v[pl.ds(k * 16, 16)] = jnp.zeros((16,), i32)

    def rank_step(k, c):
        sl = pl.ds(k * 16, 16)
        jv = jid_v[sl]
        before = plsc.load_gather(hist_v, [jv])
        cnt, last = plsc.scan_count(jv)
        rank_v[sl] = before + cnt - _SCB
        plsc.store_scatter(hist_v, [jv], before + cnt + (1 - _SCB), mask=last)
        return c

    lax.fori_loop(0, B // 16, rank_step, jnp.zeros((), i32))

    carry = jnp.zeros((), i32)
    for k in range(J // 16):
        sl = pl.ds(k * 16, 16)
        padded = ((hist_v[sl] + (R - 1)) // R) * R
        incl = plsc.cumsum(padded)
        excl = incl - padded + carry
        base_v[sl] = excl
        end_v[sl] = excl + padded
        carry = carry + jnp.sum(padded)

    @pl.when(wid == 0)
    def _():
        pltpu.sync_copy(end_v, end_hbm)

    for k in range(SB // 16):
        sl = pl.ds(k * 16, 16)
        g = wid * SB + k * 16
        jv = jid_v[pl.ds(g, 16)]
        pos_v[sl] = plsc.load_gather(base_v, [jv]) + rank_v[pl.ds(g, 16)]
    pltpu.sync_copy(pos_v, pos_hbm.at[pl.ds(wid * SB, SB)])

    pltpu.sync_copy(x_hbm.at[pl.ds(wid * SB, SB)], rows_v)
    pltpu.async_copy(rows_v, xs_hbm.at[pos_v], sem).wait()


def _unsort_body(SB, NC, osort_hbm, pos_hbm, out_hbm, pos_v, rows_v, sem):
    wid = lax.axis_index("s") * NC + lax.axis_index("c")
    base = wid * SB
    pltpu.sync_copy(pos_hbm.at[pl.ds(base, SB)], pos_v)
    pltpu.async_copy(osort_hbm.at[pos_v], rows_v, sem).wait()
    pltpu.sync_copy(rows_v, out_hbm.at[pl.ds(base, SB)])


def _segment_body(D, H1, H2, QC, x_ref, end_ref, w1c_ref, b1c_ref, w2c_ref,
                  b2c_ref, vwc_ref, vbc_ref, out_ref):
    f32 = jnp.float32
    gg = pl.program_id(0)
    ends = end_ref[...]
    gi = lax.broadcasted_iota(jnp.int32, (QC, QC), 0) // 5
    gj = lax.broadcasted_iota(jnp.int32, (QC, QC), 1) // 5
    grp = (gi == gj).astype(f32)

    bjs = []
    for sb in range(G):
        s = (gg * G + sb) * R
        bjs.append(jnp.minimum(jnp.sum((ends <= s).astype(jnp.int32)),
                               end_ref.shape[1] - 1))
    z1s = []
    for sb in range(G):
        w1 = w1c_ref[bjs[sb]]
        x = x_ref[sb * R:(sb + 1) * R]
        z1s.append(jnp.maximum(
            lax.dot_general(x, w1[:, :D], _DN_T, preferred_element_type=f32)
            + b1c_ref[bjs[sb]], 0.0))
    z2s = []
    for sb in range(G):
        w2 = w2c_ref[bjs[sb]]
        z2s.append(jnp.maximum(
            lax.dot_general(z1s[sb], w2[:, :H1], _DN_T,
                            preferred_element_type=f32)
            + b2c_ref[bjs[sb]], 0.0))
    lgs = []
    for sb in range(G):
        vw = vwc_ref[bjs[sb]]
        lgs.append(lax.dot_general(z2s[sb], vw[:, :H2], _DN_T,
                                   preferred_element_type=f32)
                   + vbc_ref[bjs[sb]])
    for sb in range(G):
        lg = lgs[sb]
        mx = jnp.max(lg, axis=1, keepdims=True)
        e = jnp.exp(lg - mx)
        denom = jnp.dot(e, grp, preferred_element_type=f32)
        out_ref[sb * R:(sb + 1) * R, :QC] = e / denom


def kernel(x, judge_ids, W1_w, W1_b, W2_w, W2_b, W1a_w, W1a_b, W2a_w, W2a_b,
           V_w, V_b, Va_w, Va_b):
    B, D = x.shape
    J, H1, _ = W1a_w.shape
    H2 = W2a_w.shape[1]
    Q, C, _ = V_w.shape
    QC = Q * C
    Bpad = 2 * B
    NB = Bpad // R

    info = plsc.get_sparse_core_info()
    NC, NS = info.num_cores, info.num_subcores
    NW = NC * NS
    SB = B // NW
    i32 = jnp.int32
    f32 = jnp.float32

    jid = judge_ids.astype(i32)
    w1c = W1_w[None] + W1a_w
    w2c = W2_w[None] + W2a_w
    vwc = (V_w.reshape(QC, H2 + 1)[None]
           + Va_w.reshape(J, QC, H2 + 1))
    b1c = ((W1_b + W1_w[:, D])[None]
           + W1a_b + W1a_w[:, :, D]).reshape(J, 1, H1)
    b2c = ((W2_b + W2_w[:, H1])[None]
           + W2a_b + W2a_w[:, :, H1]).reshape(J, 1, H2)
    vb = V_b.reshape(QC) + V_w.reshape(QC, H2 + 1)[:, H2]
    vbc = (vb[None] + Va_b.reshape(J, QC)
           + Va_w.reshape(J, QC, H2 + 1)[:, :, H2]).reshape(J, 1, QC)

    mesh = plsc.VectorSubcoreMesh(core_axis_name="c", subcore_axis_name="s")

    dispatch = pl.kernel(
        functools.partial(_dispatch_body, B, J, SB, NW, NC),
        out_type=(jax.ShapeDtypeStruct((Bpad, D), f32),
                  jax.ShapeDtypeStruct((B,), i32),
                  jax.ShapeDtypeStruct((J,), i32)),
        mesh=mesh,
        compiler_params=pltpu.CompilerParams(needs_layout_passes=False),
        scratch_types=[
            pltpu.VMEM((B,), i32),
            pltpu.VMEM((J,), i32),
            pltpu.VMEM((B,), i32),
            pltpu.VMEM((J,), i32),
            pltpu.VMEM((J,), i32),
            pltpu.VMEM((SB,), i32),
            pltpu.VMEM((SB, D), f32),
            pltpu.SemaphoreType.DMA,
        ],
    )
    x_sorted, pos, end = dispatch(jid, x)

    out_sorted = pl.pallas_call(
        functools.partial(_segment_body, D, H1, H2, QC),
        grid=(NB // G,),
        in_specs=[
            pl.BlockSpec((G * R, D), lambda g: (g, 0)),
            pl.BlockSpec((1, J), lambda g: (0, 0)),
            pl.BlockSpec((J, H1, D + 1), lambda g: (0, 0, 0)),
            pl.BlockSpec((J, 1, H1), lambda g: (0, 0, 0)),
            pl.BlockSpec((J, H2, H1 + 1), lambda g: (0, 0, 0)),
            pl.BlockSpec((J, 1, H2), lambda g: (0, 0, 0)),
            pl.BlockSpec((J, QC, H2 + 1), lambda g: (0, 0, 0)),
            pl.BlockSpec((J, 1, QC), lambda g: (0, 0, 0)),
        ],
        out_specs=pl.BlockSpec((G * R, OUTW), lambda g: (g, 0)),
        out_shape=jax.ShapeDtypeStruct((Bpad, OUTW), f32),
    )(x_sorted, end.reshape(1, J), w1c, b1c, w2c, b2c, vwc, vbc)

    unsort = pl.kernel(
        functools.partial(_unsort_body, SB, NC),
        out_type=jax.ShapeDtypeStruct((B, OUTW), f32),
        mesh=mesh,
        compiler_params=pltpu.CompilerParams(needs_layout_passes=False),
        scratch_types=[
            pltpu.VMEM((SB,), i32),
            pltpu.VMEM((SB, OUTW), f32),
            pltpu.SemaphoreType.DMA,
        ],
    )
    out_full = unsort(out_sorted, pos)
    return out_full[:, :QC].reshape(B, Q, C).transpose(1, 0, 2)


# speedup vs baseline: 12.9306x; 1.2535x over previous
import functools

import jax
import jax.numpy as jnp
from jax import lax
from jax.experimental import pallas as pl
from jax.experimental.pallas import tpu as pltpu
from jax.experimental.pallas import tpu_sc as plsc

R = 64
G = 16
OUTW = 128
_SCB = 1

_DN_T = (((1,), (1,)), ((), ()))


def _dispatch_body(B, J, SB, NW, NC, jid_hbm, x_hbm,
                   xs_hbm, pos_hbm, end_hbm,
                   jid_v, hist_v, rank_v, base_v, end_v,
                   pos_v, rows_v, sem):
    wid = lax.axis_index("s") * NC + lax.axis_index("c")
    i32 = jnp.int32

    pltpu.sync_copy(jid_hbm, jid_v)

    for k in range(J // 16):
        hist_v[pl.ds(k * 16, 16)] = jnp.zeros((16,), i32)

    def rank_step(k, c):
        sl = pl.ds(k * 16, 16)
        jv = jid_v[sl]
        before = plsc.load_gather(hist_v, [jv])
        cnt, last = plsc.scan_count(jv)
        rank_v[sl] = before + cnt - _SCB
        plsc.store_scatter(hist_v, [jv], before + cnt + (1 - _SCB), mask=last)
        return c

    lax.fori_loop(0, B // 16, rank_step, jnp.zeros((), i32))

    carry = jnp.zeros((), i32)
    for k in range(J // 16):
        sl = pl.ds(k * 16, 16)
        padded = ((hist_v[sl] + (R - 1)) // R) * R
        incl = plsc.cumsum(padded)
        excl = incl - padded + carry
        base_v[sl] = excl
        end_v[sl] = excl + padded
        carry = carry + jnp.sum(padded)

    @pl.when(wid == 0)
    def _():
        pltpu.sync_copy(end_v, end_hbm)

    for k in range(SB // 16):
        sl = pl.ds(k * 16, 16)
        g = wid * SB + k * 16
        jv = jid_v[pl.ds(g, 16)]
        pos_v[sl] = plsc.load_gather(base_v, [jv]) + rank_v[pl.ds(g, 16)]
    pltpu.sync_copy(pos_v, pos_hbm.at[pl.ds(wid * SB, SB)])

    pltpu.sync_copy(x_hbm.at[pl.ds(wid * SB, SB)], rows_v)
    pltpu.async_copy(rows_v, xs_hbm.at[pos_v], sem).wait()


def _unsort_body(SB, NC, osort_hbm, pos_hbm, out_hbm, pos_v, rows_v, sem):
    wid = lax.axis_index("s") * NC + lax.axis_index("c")
    base = wid * SB
    pltpu.sync_copy(pos_hbm.at[pl.ds(base, SB)], pos_v)
    pltpu.async_copy(osort_hbm.at[pos_v], rows_v, sem).wait()
    pltpu.sync_copy(rows_v, out_hbm.at[pl.ds(base, SB)])


def _segment_body(D, H1, H2, QC, x_ref, end_ref, w1s_ref, w1a_ref, b1c_ref,
                  w2s_ref, w2a_ref, b2c_ref, vws_ref, vwa_ref, vbc_ref,
                  out_ref):
    f32 = jnp.float32
    gg = pl.program_id(0)
    ends = end_ref[...]
    gi = lax.broadcasted_iota(jnp.int32, (QC, QC), 0) // 5
    gj = lax.broadcasted_iota(jnp.int32, (QC, QC), 1) // 5
    grp = (gi == gj).astype(f32)
    ws1 = w1s_ref[...]
    ws2 = w2s_ref[...]
    wsv = vws_ref[...]

    bjs = []
    for sb in range(G):
        s = (gg * G + sb) * R
        bjs.append(jnp.minimum(jnp.sum((ends <= s).astype(jnp.int32)),
                               end_ref.shape[1] - 1))
    z1s = []
    for sb in range(G):
        w1 = ws1 + w1a_ref[bjs[sb]]
        x = x_ref[sb * R:(sb + 1) * R]
        z1s.append(jnp.maximum(
            lax.dot_general(x, w1[:, :D], _DN_T, preferred_element_type=f32)
            + b1c_ref[bjs[sb]], 0.0))
    z2s = []
    for sb in range(G):
        w2 = ws2 + w2a_ref[bjs[sb]]
        z2s.append(jnp.maximum(
            lax.dot_general(z1s[sb], w2[:, :H1], _DN_T,
                            preferred_element_type=f32)
            + b2c_ref[bjs[sb]], 0.0))
    lgs = []
    for sb in range(G):
        vw = wsv + vwa_ref[bjs[sb]]
        lgs.append(lax.dot_general(z2s[sb], vw[:, :H2], _DN_T,
                                   preferred_element_type=f32)
                   + vbc_ref[bjs[sb]])
    for sb in range(G):
        lg = lgs[sb]
        mx = jnp.max(lg, axis=1, keepdims=True)
        e = jnp.exp(lg - mx)
        denom = jnp.dot(e, grp, preferred_element_type=f32)
        out_ref[sb * R:(sb + 1) * R, :QC] = e / denom


def kernel(x, judge_ids, W1_w, W1_b, W2_w, W2_b, W1a_w, W1a_b, W2a_w, W2a_b,
           V_w, V_b, Va_w, Va_b):
    B, D = x.shape
    J, H1, _ = W1a_w.shape
    H2 = W2a_w.shape[1]
    Q, C, _ = V_w.shape
    QC = Q * C
    Bpad = 2 * B
    NB = Bpad // R

    info = plsc.get_sparse_core_info()
    NC, NS = info.num_cores, info.num_subcores
    NW = NC * NS
    SB = B // NW
    i32 = jnp.int32
    f32 = jnp.float32

    jid = judge_ids.astype(i32)
    vws = V_w.reshape(QC, H2 + 1)
    vwa = Va_w.reshape(J, QC, H2 + 1)
    b1c = ((W1_b + W1_w[:, D])[None]
           + W1a_b + W1a_w[:, :, D]).reshape(J, 1, H1)
    b2c = ((W2_b + W2_w[:, H1])[None]
           + W2a_b + W2a_w[:, :, H1]).reshape(J, 1, H2)
    vb = V_b.reshape(QC) + vws[:, H2]
    vbc = (vb[None] + Va_b.reshape(J, QC)
           + vwa[:, :, H2]).reshape(J, 1, QC)

    mesh = plsc.VectorSubcoreMesh(core_axis_name="c", subcore_axis_name="s")

    dispatch = pl.kernel(
        functools.partial(_dispatch_body, B, J, SB, NW, NC),
        out_type=(jax.ShapeDtypeStruct((Bpad, D), f32),
                  jax.ShapeDtypeStruct((B,), i32),
                  jax.ShapeDtypeStruct((J,), i32)),
        mesh=mesh,
        compiler_params=pltpu.CompilerParams(needs_layout_passes=False),
        scratch_types=[
            pltpu.VMEM((B,), i32),
            pltpu.VMEM((J,), i32),
            pltpu.VMEM((B,), i32),
            pltpu.VMEM((J,), i32),
            pltpu.VMEM((J,), i32),
            pltpu.VMEM((SB,), i32),
            pltpu.VMEM((SB, D), f32),
            pltpu.SemaphoreType.DMA,
        ],
    )
    x_sorted, pos, end = dispatch(jid, x)

    out_sorted = pl.pallas_call(
        functools.partial(_segment_body, D, H1, H2, QC),
        grid=(NB // G,),
        in_specs=[
            pl.BlockSpec((G * R, D), lambda g: (g, 0)),
            pl.BlockSpec((1, J), lambda g: (0, 0)),
            pl.BlockSpec((H1, D + 1), lambda g: (0, 0)),
            pl.BlockSpec((J, H1, D + 1), lambda g: (0, 0, 0)),
            pl.BlockSpec((J, 1, H1), lambda g: (0, 0, 0)),
            pl.BlockSpec((H2, H1 + 1), lambda g: (0, 0)),
            pl.BlockSpec((J, H2, H1 + 1), lambda g: (0, 0, 0)),
            pl.BlockSpec((J, 1, H2), lambda g: (0, 0, 0)),
            pl.BlockSpec((QC, H2 + 1), lambda g: (0, 0)),
            pl.BlockSpec((J, QC, H2 + 1), lambda g: (0, 0, 0)),
            pl.BlockSpec((J, 1, QC), lambda g: (0, 0, 0)),
        ],
        out_specs=pl.BlockSpec((G * R, OUTW), lambda g: (g, 0)),
        out_shape=jax.ShapeDtypeStruct((Bpad, OUTW), f32),
    )(x_sorted, end.reshape(1, J), W1_w, W1a_w, b1c,
      W2_w, W2a_w, b2c, vws, vwa, vbc)

    unsort = pl.kernel(
        functools.partial(_unsort_body, SB, NC),
        out_type=jax.ShapeDtypeStruct((B, OUTW), f32),
        mesh=mesh,
        compiler_params=pltpu.CompilerParams(needs_layout_passes=False),
        scratch_types=[
            pltpu.VMEM((SB,), i32),
            pltpu.VMEM((SB, OUTW), f32),
            pltpu.SemaphoreType.DMA,
        ],
    )
    out_full = unsort(out_sorted, pos)
    return out_full[:, :QC].reshape(B, Q, C).transpose(1, 0, 2)
